# Initial kernel scaffold; baseline (speedup 1.0000x reference)
import jax
import jax.numpy as jnp
from jax.experimental import pallas as pl


def kernel(x_zone, x_shelter, edge_index_zz, edge_index_sz, gvec, mask, Wp_zone, bp_zone, Wp_shelter, bp_shelter, W0zz, as0zz, ad0zz, b0zz, W0sz, as0sz, ad0sz, b0sz, W1zz, as1zz, ad1zz, b1zz, W1sz, as1sz, ad1sz, b1sz, Wa1, ba1, Wa2, ba2, Wc1, bc1, Wc2, bc2):
    raise NotImplementedError("write your pallas kernel here")


# speedup vs baseline: 170.4372x; 170.4372x over previous
import functools

import jax
import jax.numpy as jnp
from jax import lax
from jax.experimental import pallas as pl
from jax.experimental.pallas import tpu as pltpu
from jax.experimental.pallas import tpu_sc as plsc

H = 2
C = 32
HID = 64
N = 50000
NPAD = 51200
RBLK = 2048
GRID = NPAD // RBLK
E = 800000
NSUB = 16
EPT = E // NSUB
STEP = 256
NPAIR = 97
NSTEP = 2 * NPAIR
TAIL = EPT - NSTEP * STEP
GRP = 128
NACC = 50048
ROWS_PT = NACC // NSUB


def _dot(a, b):
    return jnp.dot(a, b, preferred_element_type=jnp.float32,
                   precision=lax.Precision.HIGHEST)


_XSPEC = pl.BlockSpec((RBLK, HID), lambda i: (i, 0))
_HSPEC = pl.BlockSpec((RBLK, H, C), lambda i: (i, 0, 0))
_LSPEC = pl.BlockSpec((RBLK, 8), lambda i: (i, 0))
_NSPEC = pl.BlockSpec((H, RBLK, C), lambda i: (0, i, 0))
_DSPEC = pl.BlockSpec((H, RBLK), lambda i: (0, i))
_BSPEC = pl.BlockSpec((1, HID), lambda i: (0, 0))


def _wspec(k):
    return pl.BlockSpec((k, HID), lambda i: (0, 0))


_ASPEC = pl.BlockSpec((HID, 8), lambda i: (0, 0))

_HOUT = jax.ShapeDtypeStruct((NPAD, H, C), jnp.float32)
_LOUT = jax.ShapeDtypeStruct((NPAD, 8), jnp.float32)
_XOUT = jax.ShapeDtypeStruct((NPAD, HID), jnp.float32)


def _proj2_body(xz_ref, wz_ref, bz_ref, xs_ref, ws_ref, bs_ref, z_ref, s_ref):
    z_ref[...] = _dot(xz_ref[...], wz_ref[...]) + bz_ref[...]
    s_ref[...] = _dot(xs_ref[...], ws_ref[...]) + bs_ref[...]


def _proj2(xz, wz, bz, xs, ws, bs):
    return pl.pallas_call(
        _proj2_body,
        grid=(GRID,),
        in_specs=[
            pl.BlockSpec((RBLK, 32), lambda i: (i, 0)), _wspec(32), _BSPEC,
            pl.BlockSpec((RBLK, 16), lambda i: (i, 0)), _wspec(16), _BSPEC,
        ],
        out_specs=[_XSPEC, _XSPEC],
        out_shape=[_XOUT, _XOUT],
    )(xz, wz, bz, xs, ws, bs)


def _sprep2_body(s_ref, w0_ref, a0_ref, w1_ref, a1_ref,
                 h0_ref, l0_ref, h1_ref, l1_ref):
    h0 = _dot(s_ref[...], w0_ref[...])
    h0_ref[...] = h0.reshape(RBLK, H, C)
    l0_ref[...] = _dot(h0, a0_ref[...])
    h1 = _dot(s_ref[...], w1_ref[...])
    h1_ref[...] = h1.reshape(RBLK, H, C)
    l1_ref[...] = _dot(h1, a1_ref[...])


def _sprep2(s, w0, a0, w1, a1):
    return pl.pallas_call(
        _sprep2_body,
        grid=(GRID,),
        in_specs=[_XSPEC, _wspec(HID), _ASPEC, _wspec(HID), _ASPEC],
        out_specs=[_HSPEC, _LSPEC, _HSPEC, _LSPEC],
        out_shape=[_HOUT, _LOUT, _HOUT, _LOUT],
    )(s, w0, a0, w1, a1)


def _zprep_body(z_ref, wzz_ref, azz_ref, wsz_ref, asz_ref,
                hzz_ref, lzz_ref, lszd_ref):
    hzz = _dot(z_ref[...], wzz_ref[...])
    hzz_ref[...] = hzz.reshape(RBLK, H, C)
    lzz_ref[...] = _dot(hzz, azz_ref[...])
    hsz = _dot(z_ref[...], wsz_ref[...])
    lszd_ref[...] = _dot(hsz, asz_ref[...])


def _zprep(z, wzz, azz, wsz, asz):
    return pl.pallas_call(
        _zprep_body,
        grid=(GRID,),
        in_specs=[_XSPEC, _wspec(HID), _ASPEC, _wspec(HID), _ASPEC],
        out_specs=[_HSPEC, _LSPEC, _LSPEC],
        out_shape=[_HOUT, _LOUT, _LOUT],
    )(z, wzz, azz, wsz, asz)


def _combine_block(nzz_ref, dzz_ref, nsz_ref, dsz_ref, bb_ref, i):
    eps = jnp.float32(1e-16)
    o0 = (nzz_ref[0] / (dzz_ref[0][:, None] + eps)
          + nsz_ref[0] / (dsz_ref[0][:, None] + eps))
    o1 = (nzz_ref[1] / (dzz_ref[1][:, None] + eps)
          + nsz_ref[1] / (dsz_ref[1][:, None] + eps))
    o = jnp.concatenate([o0, o1], axis=1) + bb_ref[...]
    o = jnp.maximum(o, 0.0)
    row = lax.broadcasted_iota(jnp.int32, (RBLK, 1), 0) + i * RBLK
    return jnp.where(row < N, o, 0.0)


def _czprep_body(nzz_ref, dzz_ref, nsz_ref, dsz_ref, bb_ref,
                 wzz_ref, azz_ref, wsz_ref, asz_ref,
                 hzz_ref, lzz_ref, lszd_ref):
    z = _combine_block(nzz_ref, dzz_ref, nsz_ref, dsz_ref, bb_ref,
                       pl.program_id(0))
    hzz = _dot(z, wzz_ref[...])
    hzz_ref[...] = hzz.reshape(RBLK, H, C)
    lzz_ref[...] = _dot(hzz, azz_ref[...])
    hsz = _dot(z, wsz_ref[...])
    lszd_ref[...] = _dot(hsz, asz_ref[...])


def _czprep(num_zz, den_zz, num_sz, den_sz, bb, wzz, azz, wsz, asz):
    return pl.pallas_call(
        _czprep_body,
        grid=(GRID,),
        in_specs=[_NSPEC, _DSPEC, _NSPEC, _DSPEC, _BSPEC,
                  _wspec(HID), _ASPEC, _wspec(HID), _ASPEC],
        out_specs=[_HSPEC, _LSPEC, _LSPEC],
        out_shape=[_HOUT, _LOUT, _LOUT],
    )(num_zz, den_zz, num_sz, den_sz, bb, wzz, azz, wsz, asz)


def _cmeans_body(nzz_ref, dzz_ref, nsz_ref, dsz_ref, bb_ref, s_ref,
                 so_ref, zo_ref):
    i = pl.program_id(0)
    z = _combine_block(nzz_ref, dzz_ref, nsz_ref, dsz_ref, bb_ref, i)
    row = lax.broadcasted_iota(jnp.int32, (RBLK, 1), 0) + i * RBLK
    m = (row < N).astype(jnp.float32)
    sb = jnp.sum(s_ref[...] * m, axis=0, keepdims=True)
    zb = jnp.sum(z, axis=0, keepdims=True)

    @pl.when(i == 0)
    def _():
        so_ref[...] = sb
        zo_ref[...] = zb

    @pl.when(i != 0)
    def _():
        so_ref[...] += sb
        zo_ref[...] += zb


def _cmeans(num_zz, den_zz, num_sz, den_sz, bb, s):
    return pl.pallas_call(
        _cmeans_body,
        grid=(GRID,),
        in_specs=[_NSPEC, _DSPEC, _NSPEC, _DSPEC, _BSPEC, _XSPEC],
        out_specs=[_BSPEC, _BSPEC],
        out_shape=[
            jax.ShapeDtypeStruct((1, HID), jnp.float32),
            jax.ShapeDtypeStruct((1, HID), jnp.float32),
        ],
    )(num_zz, den_zz, num_sz, den_sz, bb, s)


def _head_body(ss_ref, zs_ref, g_ref, mk_ref, wa1_ref, ba1_ref, wa2_ref,
               ba2_ref, wc1_ref, bc1_ref, wc2_ref, bc2_ref, lg_ref, vl_ref):
    inv = jnp.float32(N)
    feat = jnp.concatenate(
        [ss_ref[...] / inv, zs_ref[...] / inv, g_ref[...]], axis=1)
    a1 = jnp.dot(feat, wa1_ref[...], preferred_element_type=jnp.float32, precision=lax.Precision.HIGHEST)
    a1 = jnp.maximum(a1 + ba1_ref[...], 0.0)
    lgt = jnp.dot(a1, wa2_ref[...], preferred_element_type=jnp.float32, precision=lax.Precision.HIGHEST)
    lgt = lgt + ba2_ref[...]
    c1 = jnp.dot(feat, wc1_ref[...], preferred_element_type=jnp.float32, precision=lax.Precision.HIGHEST)
    c1 = jnp.maximum(c1 + bc1_ref[...], 0.0)
    vl = jnp.dot(c1, wc2_ref[...], preferred_element_type=jnp.float32, precision=lax.Precision.HIGHEST)
    vl_ref[...] = vl + bc2_ref[...]
    lg_ref[...] = jnp.where(mk_ref[...] != 0, lgt, -jnp.inf)


def _head(ssum, zsum, gvec, mask, wa1, ba1, wa2, ba2, wc1, bc1, wc2, bc2):
    return pl.pallas_call(
        _head_body,
        out_shape=[
            jax.ShapeDtypeStruct((1, 32), jnp.float32),
            jax.ShapeDtypeStruct((1, 1), jnp.float32),
        ],
    )(ssum, zsum, gvec, mask, wa1, ba1, wa2, ba2, wc1, bc1, wc2, bc2)


def _sc_body(ei_hbm, h_hbm, lgs_hbm, lgd_hbm, num_hbm, den_hbm,
             rowidx, aidx, didx, dst2d, dst80, va, vd, wv, rows,
             num_sp, den_sp, sem_a, sem_g, sem_s):
    c = lax.axis_index("c")
    s = lax.axis_index("s")
    ebase = s * EPT
    BUF = 2 * STEP
    row_lo = s * ROWS_PT

    def fire_a(h, base):
        ho = h * STEP
        pltpu.async_copy(ei_hbm.at[pl.ds(base, STEP)],
                         rowidx.at[pl.ds(ho, STEP)], sem_a)
        pltpu.async_copy(ei_hbm.at[pl.ds(E + base, STEP)],
                         didx.at[pl.ds(ho, STEP)], sem_a)

    def drain_a(h):
        ho = h * STEP
        pltpu.make_async_copy(ei_hbm.at[pl.ds(0, STEP)],
                              rowidx.at[pl.ds(ho, STEP)], sem_a).wait()
        pltpu.make_async_copy(ei_hbm.at[pl.ds(0, STEP)],
                              didx.at[pl.ds(ho, STEP)], sem_a).wait()

    def xform(h):
        ho = h * STEP
        for g in range(STEP // GRP):
            for jj in range(GRP // 16):
                off = ho + g * GRP + jj * 16
                sv = rowidx[pl.ds(off, 16)]
                rowidx[pl.ds(off, 16)] = sv * 2 + c
                aidx[pl.ds(off, 16)] = sv * 8 + c
                dv = didx[pl.ds(off, 16)]
                dst2d[2 * h + g, pl.ds(jj * 16, 16)] = dv
                didx[pl.ds(off, 16)] = dv * 8 + (2 + c)

    def fire_c(h):
        ho = h * STEP
        for g in range(STEP // GRP):
            sl = pl.ds(ho + g * GRP, GRP)
            pltpu.async_copy(h_hbm.at[rowidx.at[sl]], rows.at[sl, :], sem_g)
            pltpu.async_copy(lgs_hbm.at[aidx.at[sl]], va.at[sl], sem_g)
            pltpu.async_copy(lgd_hbm.at[didx.at[sl]], vd.at[sl], sem_g)

    def drain_c(h):
        ho = h * STEP
        for g in range(STEP // GRP):
            sl = pl.ds(ho + g * GRP, GRP)
            pltpu.make_async_copy(h_hbm.at[pl.ds(0, GRP)],
                                  rows.at[sl, :], sem_g).wait()
            pltpu.make_async_copy(lgs_hbm.at[pl.ds(0, GRP)],
                                  va.at[sl], sem_g).wait()
            pltpu.make_async_copy(lgd_hbm.at[pl.ds(0, GRP)],
                                  vd.at[sl], sem_g).wait()

    def compute_and_scatter(h):
        ho = h * STEP
        for off in range(ho, ho + STEP, 16):
            e = va[pl.ds(off, 16)] + vd[pl.ds(off, 16)]
            e = jnp.where(e > 0.0, e, 0.2 * e)
            wv[pl.ds(off, 16)] = jnp.exp(e)

        zero16 = jnp.zeros((16,), jnp.int32)

        def mul_body(j, carry):
            w = plsc.load_gather(wv, [zero16 + j])
            rows[j, pl.ds(0, 16)] = rows[j, pl.ds(0, 16)] * w
            rows[j, pl.ds(16, 16)] = rows[j, pl.ds(16, 16)] * w
            return carry

        for g in range(STEP // GRP):
            base = ho + g * GRP
            lax.fori_loop(base, base + GRP, mul_body, 0, unroll=8)
            sl = pl.ds(base, GRP)
            pltpu.async_copy(rows.at[sl, :],
                             num_sp.at[dst2d.at[2 * h + g]], sem_s, add=True)
            pltpu.async_copy(wv.at[sl],
                             den_sp.at[dst2d.at[2 * h + g]], sem_s, add=True)

    def drain_s(h):
        for g in range(STEP // GRP):
            sl = pl.ds(h * STEP + g * GRP, GRP)
            pltpu.make_async_copy(rows.at[sl, :],
                                  num_sp.at[pl.ds(0, GRP), :], sem_s).wait()
            pltpu.make_async_copy(wv.at[sl],
                                  den_sp.at[pl.ds(0, GRP)], sem_s).wait()

    def zrow_loop(j, carry):
        rows[j, pl.ds(0, 16)] = jnp.zeros((16,), jnp.float32)
        rows[j, pl.ds(16, 16)] = jnp.zeros((16,), jnp.float32)
        return carry

    lax.fori_loop(0, BUF, zrow_loop, 0, unroll=4)

    def zw_loop(j, carry):
        wv[pl.ds(j * 16, 16)] = jnp.zeros((16,), jnp.float32)
        return carry

    lax.fori_loop(0, BUF // 16, zw_loop, 0, unroll=4)

    zcps = []
    nfull = ROWS_PT // BUF
    for k in range(nfull):
        zcps.append(pltpu.async_copy(
            rows, num_sp.at[pl.ds(row_lo + k * BUF, BUF), :], sem_s))
        zcps.append(pltpu.async_copy(
            wv, den_sp.at[pl.ds(row_lo + k * BUF, BUF)], sem_s))
    nrem = ROWS_PT - nfull * BUF
    if nrem:
        zcps.append(pltpu.async_copy(
            rows.at[pl.ds(0, nrem), :],
            num_sp.at[pl.ds(row_lo + ROWS_PT - nrem, nrem), :], sem_s))
        zcps.append(pltpu.async_copy(
            wv.at[pl.ds(0, nrem)],
            den_sp.at[pl.ds(row_lo + ROWS_PT - nrem, nrem)], sem_s))
    for cp in zcps:
        cp.wait()
    plsc.subcore_barrier()

    fire_a(0, ebase)
    drain_a(0)
    xform(0)
    fire_a(1, ebase + STEP)
    fire_c(0)

    def pair_body(t, carry):
        drain_c(0)
        drain_a(1)

        @pl.when(t > 0)
        def _():
            drain_s(1)

        xform(1)
        fire_c(1)

        @pl.when(t < NPAIR - 1)
        def _():
            fire_a(0, ebase + (t * 2 + 2) * STEP)

        compute_and_scatter(0)

        drain_c(1)

        @pl.when(t < NPAIR - 1)
        def _():
            drain_a(0)
            drain_s(0)
            xform(0)
            fire_c(0)
            fire_a(1, ebase + (t * 2 + 3) * STEP)

        compute_and_scatter(1)
        return carry

    lax.fori_loop(0, NPAIR, pair_body, 0)
    drain_s(0)
    drain_s(1)

    tb = ebase + NSTEP * STEP
    pltpu.sync_copy(ei_hbm.at[pl.ds(tb, TAIL)], rowidx.at[pl.ds(0, TAIL)])
    pltpu.sync_copy(ei_hbm.at[pl.ds(E + tb, TAIL)], didx.at[pl.ds(0, TAIL)])
    for j in range(TAIL // 16):
        off = j * 16
        sv = rowidx[pl.ds(off, 16)]
        rowidx[pl.ds(off, 16)] = sv * 2 + c
        aidx[pl.ds(off, 16)] = sv * 8 + c
        dv = didx[pl.ds(off, 16)]
        if off + 16 <= 2 * GRP:
            g = off // GRP
            dst2d[g, pl.ds(off - g * GRP, 16)] = dv
        else:
            dst80[0, pl.ds(off - 2 * GRP, 16)] = dv
        didx[pl.ds(off, 16)] = dv * 8 + (2 + c)
    cps = []
    for base, n in ((0, GRP), (GRP, GRP), (2 * GRP, TAIL - 2 * GRP)):
        sl = pl.ds(base, n)
        cps.append(pltpu.async_copy(h_hbm.at[rowidx.at[sl]],
                                    rows.at[sl, :], sem_g))
        cps.append(pltpu.async_copy(lgs_hbm.at[aidx.at[sl]], va.at[sl], sem_g))
        cps.append(pltpu.async_copy(lgd_hbm.at[didx.at[sl]], vd.at[sl], sem_g))
    for cp in cps:
        cp.wait()
    for off in range(0, TAIL, 16):
        e = va[pl.ds(off, 16)] + vd[pl.ds(off, 16)]
        e = jnp.where(e > 0.0, e, 0.2 * e)
        wv[pl.ds(off, 16)] = jnp.exp(e)
    zero16 = jnp.zeros((16,), jnp.int32)

    def tmul_body(j, carry):
        w = plsc.load_gather(wv, [zero16 + j])
        rows[j, pl.ds(0, 16)] = rows[j, pl.ds(0, 16)] * w
        rows[j, pl.ds(16, 16)] = rows[j, pl.ds(16, 16)] * w
        return carry

    lax.fori_loop(0, TAIL, tmul_body, 0, unroll=8)
    for g, (base, n) in enumerate(((0, GRP), (GRP, GRP),
                                   (2 * GRP, TAIL - 2 * GRP))):
        sl = pl.ds(base, n)
        idxr = dst2d.at[g] if g < 2 else dst80.at[0]
        pltpu.async_copy(rows.at[sl, :], num_sp.at[idxr], sem_s,
                         add=True).wait()
        pltpu.async_copy(wv.at[sl], den_sp.at[idxr], sem_s, add=True).wait()

    plsc.subcore_barrier()

    wb1 = pltpu.async_copy(num_sp.at[pl.ds(row_lo, ROWS_PT), :],
                           num_hbm.at[c, pl.ds(row_lo, ROWS_PT), :], sem_g)
    wb2 = pltpu.async_copy(den_sp.at[pl.ds(row_lo, ROWS_PT)],
                           den_hbm.at[c, pl.ds(row_lo, ROWS_PT)], sem_g)
    wb1.wait()
    wb2.wait()


_sc_conv = pl.kernel(
    _sc_body,
    out_type=[
        jax.ShapeDtypeStruct((H, NPAD, C), jnp.float32),
        jax.ShapeDtypeStruct((H, NPAD), jnp.float32),
    ],
    mesh=plsc.VectorSubcoreMesh(core_axis_name="c", subcore_axis_name="s"),
    compiler_params=pltpu.CompilerParams(use_tc_tiling_on_sc=False,
                                         needs_layout_passes=False),
    scratch_types=[
        pltpu.VMEM((2 * STEP,), jnp.int32),
        pltpu.VMEM((2 * STEP,), jnp.int32),
        pltpu.VMEM((2 * STEP,), jnp.int32),
        pltpu.VMEM((4, GRP), jnp.int32),
        pltpu.VMEM((1, 80), jnp.int32),
        pltpu.VMEM((2 * STEP,), jnp.float32),
        pltpu.VMEM((2 * STEP,), jnp.float32),
        pltpu.VMEM((2 * STEP,), jnp.float32),
        pltpu.VMEM((2 * STEP, C), jnp.float32),
        pltpu.VMEM_SHARED((NACC, C), jnp.float32),
        pltpu.VMEM_SHARED((NACC,), jnp.float32),
        pltpu.SemaphoreType.DMA,
        pltpu.SemaphoreType.DMA,
        pltpu.SemaphoreType.DMA,
    ],
)


def _amat(a_s, a_d):
    am = jnp.zeros((HID, 8), jnp.float32)
    am = am.at[0:C, 0].set(a_s[0])
    am = am.at[C:HID, 1].set(a_s[1])
    am = am.at[0:C, 2].set(a_d[0])
    am = am.at[C:HID, 3].set(a_d[1])
    return am


def kernel(x_zone, x_shelter, edge_index_zz, edge_index_sz, gvec, mask,
           Wp_zone, bp_zone, Wp_shelter, bp_shelter,
           W0zz, as0zz, ad0zz, b0zz, W0sz, as0sz, ad0sz, b0sz,
           W1zz, as1zz, ad1zz, b1zz, W1sz, as1sz, ad1sz, b1sz,
           Wa1, ba1, Wa2, ba2, Wc1, bc1, Wc2, bc2):
    pad = NPAD - N
    xz = jnp.pad(x_zone, ((0, pad), (0, 0)))
    xs = jnp.pad(x_shelter, ((0, pad), (0, 0)))
    ei_zz = edge_index_zz.astype(jnp.int32).reshape(2 * E)
    ei_sz = edge_index_sz.astype(jnp.int32).reshape(2 * E)
    a0zz, a0sz = _amat(as0zz, ad0zz), _amat(as0sz, ad0sz)
    a1zz, a1sz = _amat(as1zz, ad1zz), _amat(as1sz, ad1sz)
    bb0 = (b0zz + b0sz).reshape(1, HID)
    bb1 = (b1zz + b1sz).reshape(1, HID)

    z0, s = _proj2(xz, Wp_zone, bp_zone.reshape(1, HID),
                   xs, Wp_shelter, bp_shelter.reshape(1, HID))
    hs0, ls0, hs1, ls1 = _sprep2(s, W0sz, a0sz, W1sz, a1sz)

    h_zz, lg_zz, lg_szd = _zprep(z0, W0zz, a0zz, W0sz, a0sz)
    num_zz0, den_zz0 = _sc_conv(ei_zz, h_zz.reshape(2 * NPAD, C),
                                lg_zz.reshape(NPAD * 8),
                                lg_zz.reshape(NPAD * 8))
    num_sz0, den_sz0 = _sc_conv(ei_sz, hs0.reshape(2 * NPAD, C),
                                ls0.reshape(NPAD * 8),
                                lg_szd.reshape(NPAD * 8))

    h_zz1, lg_zz1, lg_szd1 = _czprep(num_zz0, den_zz0, num_sz0, den_sz0,
                                     bb0, W1zz, a1zz, W1sz, a1sz)
    num_zz1, den_zz1 = _sc_conv(ei_zz, h_zz1.reshape(2 * NPAD, C),
                                lg_zz1.reshape(NPAD * 8),
                                lg_zz1.reshape(NPAD * 8))
    num_sz1, den_sz1 = _sc_conv(ei_sz, hs1.reshape(2 * NPAD, C),
                                ls1.reshape(NPAD * 8),
                                lg_szd1.reshape(NPAD * 8))

    ssum, zsum = _cmeans(num_zz1, den_zz1, num_sz1, den_sz1, bb1, s)
    lg, vl = _head(ssum, zsum, gvec.reshape(1, 6),
                   mask.astype(jnp.int32).reshape(1, 32),
                   Wa1, ba1.reshape(1, HID), Wa2, ba2.reshape(1, 32),
                   Wc1, bc1.reshape(1, HID), Wc2, bc2.reshape(1, 1))
    return lg.reshape(32), vl[0, 0]
